# baseline (speedup 1.0000x reference)
import functools

import jax
import jax.numpy as jnp
from jax import lax
from jax.experimental import pallas as pl
from jax.experimental.pallas import tpu as pltpu
from jax.experimental.pallas import tpu_sc as plsc


NC = 2
NS = 16
NW = NC * NS
CH = 64


def _sc_deg_body(n_out, nchd, dstp, ewp, out, idx_d, ew_v, zbuf, deg_sh):
    c = lax.axis_index("c")
    s = lax.axis_index("s")
    wid = s * NC + c

    pltpu.sync_copy(dstp.at[wid], idx_d)
    pltpu.sync_copy(ewp.at[wid], ew_v)

    @pl.when(s == 0)
    def _():
        z16 = jnp.zeros((16,), jnp.float32)

        @pl.loop(0, 128)
        def _z(i):
            zbuf[pl.ds(i * 16, 16)] = z16

        @pl.loop(0, n_out // 2048)
        def _zz(i):
            pltpu.sync_copy(zbuf, deg_sh.at[pl.ds(i * 2048, 2048)])

    plsc.subcore_barrier()

    @pl.loop(0, nchd)
    def _edges(ch):
        pltpu.sync_copy(ew_v.at[ch], deg_sh.at[idx_d.at[ch]], add=True)

    plsc.subcore_barrier()

    @pl.when(s == 0)
    def _():
        pltpu.sync_copy(deg_sh, out.at[c])


def _sc_deg(dstp, ewp, n_nodes):
    n_out = -(-n_nodes // 2048) * 2048
    nchd = dstp.shape[1]
    body = functools.partial(_sc_deg_body, n_out, nchd)
    return pl.kernel(
        body,
        out_type=jax.ShapeDtypeStruct((NC, n_out), jnp.float32),
        mesh=plsc.VectorSubcoreMesh(core_axis_name="c", subcore_axis_name="s",
                                    num_cores=NC, num_subcores=NS),
        scratch_types=[
            pltpu.VMEM((nchd, 128), jnp.int32),
            pltpu.VMEM((nchd, 128), jnp.float32),
            pltpu.VMEM((2048,), jnp.float32),
            pltpu.VMEM_SHARED((n_out,), jnp.float32),
        ],
    )(dstp, ewp)


def _sc_edge_body(n_pad, pw, hs, comb, ewp, out, *refs):
    nch = pw // CH
    c = lax.axis_index("c")
    s = lax.axis_index("s")
    wid = s * NC + c
    boff = wid * (pw // CH) * 8
    base = wid * pw
    sets = refs[0:8]
    ews = refs[8:16]
    rows = refs[16:20]
    sis = refs[20:28]
    sgs = refs[28:32]
    sss = refs[32:36]
    agg_sh = refs[36]
    r0 = rows[0]

    rpt = n_pad // NS
    zeros16 = jnp.zeros((16,), jnp.float32)
    for e in range(8):
        for k in range(8):
            r0[e, pl.ds(k * 16, 16)] = zeros16

    @pl.loop(0, rpt // 8)
    def _zero(i):
        pltpu.sync_copy(r0.at[pl.ds(0, 8)],
                        agg_sh.at[pl.ds(s * rpt + i * 8, 8)])

    plsc.subcore_barrier()

    def stage_descs(ch, m):
        return [
            pltpu.make_async_copy(comb.at[pl.ds(boff + ch * 8, 8)],
                                  sets[m], sis[m]),
            pltpu.make_async_copy(ewp.at[pl.ds(base + ch * CH, CH)],
                                  ews[m], sis[m]),
        ]

    def stage(ch, m):
        for d in stage_descs(ch, m):
            d.start()

    def wait_stage(ch, m):
        for d in stage_descs(ch, m):
            d.wait()

    def gather(m, b):
        pltpu.make_async_copy(hs.at[sets[m].at[0]], rows[b], sgs[b]).start()

    def wait_gather(m, b):
        pltpu.make_async_copy(hs.at[sets[m].at[0]], rows[b], sgs[b]).wait()

    def scale(m, b):
        buf = rows[b]
        ewb = ews[m]

        @pl.loop(0, CH // 16)
        def _h(h):
            wv = ewb[pl.ds(h * 16, 16)]
            for i in range(16):
                w = wv[i]
                for k in range(8):
                    sl = buf[h * 16 + i, pl.ds(k * 16, 16)]
                    buf[h * 16 + i, pl.ds(k * 16, 16)] = sl * w

    def scatter(m, b):
        pltpu.make_async_copy(rows[b], agg_sh.at[sets[m].at[1]],
                              sss[b]).start(add=True)

    def wait_scatter(m, b):
        pltpu.make_async_copy(rows[b], agg_sh.at[sets[m].at[1]],
                              sss[b]).wait()

    def body(ch, m):
        b = m % 4
        b2 = (m + 2) % 4
        m2 = (m + 2) % 8
        m4 = (m + 4) % 8
        m6 = (m + 6) % 8

        @pl.when(jnp.logical_and(ch >= 2, ch + 2 < nch))
        def _():
            wait_scatter(m6, b2)

        @pl.when(ch + 2 < nch)
        def _():
            wait_stage(ch + 2, m2)
            gather(m2, b2)

        @pl.when(ch + 4 < nch)
        def _():
            stage(ch + 4, m4)

        wait_gather(m, b)
        scale(m, b)
        scatter(m, b)

    for ch0 in range(4):
        stage(ch0, ch0)
    wait_stage(0, 0)
    gather(0, 0)
    wait_stage(1, 1)
    gather(1, 1)

    @pl.loop(0, nch, step=8)
    def _main(j):
        for m in range(8):
            body(j + m, m)

    for t in range(4, 0, -1):
        wait_scatter((nch - t) % 8, (nch - t) % 4)

    plsc.subcore_barrier()

    pltpu.sync_copy(agg_sh.at[pl.ds(s * rpt, rpt)],
                    out.at[c].at[pl.ds(s * rpt, rpt)])


def _sc_edge(hs, comb, ewp, pw):
    n_pad = -(-hs.shape[0] // (NS * 8)) * (NS * 8)
    body = functools.partial(_sc_edge_body, n_pad, pw)
    set_t = [pltpu.VMEM((8, CH), jnp.int32)] * 8
    ew_t = [pltpu.VMEM((CH,), jnp.float32)] * 8
    row_t = [pltpu.VMEM((CH, 128), jnp.float32)] * 4
    sem_t = [pltpu.SemaphoreType.DMA] * 16
    return pl.kernel(
        body,
        out_type=jax.ShapeDtypeStruct((NC, n_pad, 128), jnp.float32),
        mesh=plsc.VectorSubcoreMesh(core_axis_name="c", subcore_axis_name="s",
                                    num_cores=NC, num_subcores=NS),
        scratch_types=set_t + ew_t + row_t + sem_t + [
            pltpu.VMEM_SHARED((n_pad, 128), jnp.float32),
        ],
    )(hs, comb, ewp)


def _enc_body(z_ref, w1_ref, b1_ref, w2_ref, b2_ref, out_ref):
    h = jnp.dot(z_ref[...], w1_ref[...], preferred_element_type=jnp.float32)
    h = jnp.maximum(h + b1_ref[...], 0.0)
    out_ref[...] = (
        jnp.dot(h, w2_ref[...], preferred_element_type=jnp.float32)
        + b2_ref[...]
    )


def _tc_enc(z, w1, b1, w2, b2):
    return pl.pallas_call(
        _enc_body,
        out_shape=jax.ShapeDtypeStruct((z.shape[0], w2.shape[1]), jnp.float32),
    )(z, w1, b1, w2, b2)


def _dense1_body(blk, x_ref, dp_ref, hs_ref, dv_ref):
    i = pl.program_id(0)
    dp = dp_ref[:, pl.ds(i * blk, blk)]
    d = 1.0 + jnp.sum(dp, axis=0)
    dinv = lax.rsqrt(d)[:, None]
    rep = jnp.broadcast_to(dinv, x_ref.shape)
    dv_ref[...] = rep
    hs_ref[...] = x_ref[...] * rep


def _tc_dense1(x, deg_part, blk):
    n = x.shape[0]
    grid = -(-n // blk)
    return pl.pallas_call(
        functools.partial(_dense1_body, blk),
        grid=(grid,),
        in_specs=[
            pl.BlockSpec((blk, 128), lambda i: (i, 0)),
            pl.BlockSpec(deg_part.shape, lambda i: (0, 0)),
        ],
        out_specs=[
            pl.BlockSpec((blk, 128), lambda i: (i, 0)),
            pl.BlockSpec((blk, 128), lambda i: (i, 0)),
        ],
        out_shape=[
            jax.ShapeDtypeStruct((n, 128), jnp.float32),
            jax.ShapeDtypeStruct((n, 128), jnp.float32),
        ],
    )(x, deg_part)


def _ln(y, g_ref, b_ref):
    m = jnp.mean(y, axis=-1, keepdims=True)
    cdev = y - m
    v = jnp.mean(cdev * cdev, axis=-1, keepdims=True)
    return cdev * lax.rsqrt(v + 1e-5) * g_ref[...] + b_ref[...]


def _dense2_body(agg_ref, hs_ref, x_ref, dv_ref, w_ref, b_ref, g_ref,
                 bl_ref, h2_ref, hs2_ref):
    a = (agg_ref[0] + agg_ref[1] + hs_ref[...]) * dv_ref[...]
    y = jnp.dot(a, w_ref[...], preferred_element_type=jnp.float32) + b_ref[...]
    y = jnp.maximum(_ln(y, g_ref, bl_ref), 0.0)
    h2 = y + x_ref[...]
    h2_ref[...] = h2
    hs2_ref[...] = h2 * dv_ref[...]


def _tc_dense2(agg, hs, x, dv, w, b, g, bl, blk):
    n = x.shape[0]
    grid = -(-n // blk)
    return pl.pallas_call(
        _dense2_body,
        grid=(grid,),
        in_specs=[
            pl.BlockSpec((NC, blk, 128), lambda i: (0, i, 0)),
            pl.BlockSpec((blk, 128), lambda i: (i, 0)),
            pl.BlockSpec((blk, 128), lambda i: (i, 0)),
            pl.BlockSpec((blk, 128), lambda i: (i, 0)),
            pl.BlockSpec((128, 128), lambda i: (0, 0)),
            pl.BlockSpec((1, 128), lambda i: (0, 0)),
            pl.BlockSpec((1, 128), lambda i: (0, 0)),
            pl.BlockSpec((1, 128), lambda i: (0, 0)),
        ],
        out_specs=[
            pl.BlockSpec((blk, 128), lambda i: (i, 0)),
            pl.BlockSpec((blk, 128), lambda i: (i, 0)),
        ],
        out_shape=[
            jax.ShapeDtypeStruct((n, 128), jnp.float32),
            jax.ShapeDtypeStruct((n, 128), jnp.float32),
        ],
    )(agg, hs, x, dv, w, b, g, bl)


def _dense3_body(n, blk, grid, agg_ref, hs_ref, h_ref, dv_ref, w_ref, b_ref,
                 g_ref, bl_ref, z_ref, wt_ref, wb_ref, ab_ref, out_ref,
                 gsum_ref):
    i = pl.program_id(0)
    a = (agg_ref[0] + agg_ref[1] + hs_ref[...]) * dv_ref[...]
    y = jnp.dot(a, w_ref[...], preferred_element_type=jnp.float32) + b_ref[...]
    gblk = _ln(y, g_ref, bl_ref) + h_ref[...]
    rid = lax.broadcasted_iota(jnp.int32, gblk.shape, 0)
    gblk = jnp.where(rid < n - i * blk, gblk, 0.0)

    @pl.when(i == 0)
    def _():
        gsum_ref[...] = jnp.zeros_like(gsum_ref)

    gsum_ref[...] += jnp.sum(gblk, axis=0, keepdims=True)

    @pl.when(i == grid - 1)
    def _():
        gm = gsum_ref[...] * (1.0 / n)
        gterm = jnp.dot(gm, wb_ref[...], preferred_element_type=jnp.float32)
        out_ref[...] = (
            jnp.dot(z_ref[...], wt_ref[...],
                    preferred_element_type=jnp.float32)
            + gterm + ab_ref[...]
        )


def _tc_dense3(agg, hs, h, dv, w, b, g, bl, z_enc, wt, wb, ab, blk):
    n = h.shape[0]
    grid = -(-n // blk)
    nb = z_enc.shape[0]
    return pl.pallas_call(
        functools.partial(_dense3_body, n, blk, grid),
        grid=(grid,),
        in_specs=[
            pl.BlockSpec((NC, blk, 128), lambda i: (0, i, 0)),
            pl.BlockSpec((blk, 128), lambda i: (i, 0)),
            pl.BlockSpec((blk, 128), lambda i: (i, 0)),
            pl.BlockSpec((blk, 128), lambda i: (i, 0)),
            pl.BlockSpec((128, 128), lambda i: (0, 0)),
            pl.BlockSpec((1, 128), lambda i: (0, 0)),
            pl.BlockSpec((1, 128), lambda i: (0, 0)),
            pl.BlockSpec((1, 128), lambda i: (0, 0)),
            pl.BlockSpec((nb, 128), lambda i: (0, 0)),
            pl.BlockSpec((128, 128), lambda i: (0, 0)),
            pl.BlockSpec((128, 128), lambda i: (0, 0)),
            pl.BlockSpec((1, 128), lambda i: (0, 0)),
        ],
        out_specs=pl.BlockSpec((nb, 128), lambda i: (0, 0)),
        out_shape=jax.ShapeDtypeStruct((nb, 128), jnp.float32),
        scratch_shapes=[pltpu.VMEM((1, 128), jnp.float32)],
    )(agg, hs, h, dv, w, b, g, bl, z_enc, wt, wb, ab)


def kernel(z, x, edge_index, edge_attr, enc_W1, enc_b1, enc_W2, enc_b2,
           gcn_W1, gcn_b1, ln_g1, ln_b1, gcn_W2, gcn_b2, ln_g2, ln_b2,
           aggr_W, aggr_b):
    n = x.shape[0]
    e = edge_index.shape[1]

    pw = -(-e // NW)
    pw = -(-pw // (8 * CH)) * (8 * CH)
    ep = pw * NW
    nch = pw // CH
    pad = ep - e

    src = edge_index[0].astype(jnp.int32)
    dst = edge_index[1].astype(jnp.int32)
    pidx = (jnp.arange(pad, dtype=jnp.int32) * 97) % n
    srcp = jnp.concatenate([src, pidx])
    dstp = jnp.concatenate([dst, pidx])
    ewp = jnp.concatenate(
        [edge_attr.astype(jnp.float32), jnp.zeros((pad,), jnp.float32)]
    )
    comb = jnp.stack([srcp.reshape(-1, CH), dstp.reshape(-1, CH)], axis=1)
    comb = jnp.concatenate(
        [comb, jnp.zeros((comb.shape[0], 6, CH), jnp.int32)], axis=1
    ).reshape(-1, CH)

    pwd = -(-e // NW)
    pwd = -(-pwd // 128) * 128
    padd = pwd * NW - e
    pidxd = (jnp.arange(padd, dtype=jnp.int32) * 97) % n
    dstpd = jnp.concatenate([dst, pidxd]).reshape(NW, pwd // 128, 128)
    ewpd = jnp.concatenate(
        [edge_attr.astype(jnp.float32), jnp.zeros((padd,), jnp.float32)]
    ).reshape(NW, pwd // 128, 128)

    blk = 2048

    z_enc = _tc_enc(z, enc_W1, enc_b1.reshape(1, -1), enc_W2,
                    enc_b2.reshape(1, -1))

    deg_part = _sc_deg(dstpd, ewpd, n)
    hs1, dv = _tc_dense1(x, deg_part, blk)

    agg1 = _sc_edge(hs1, comb, ewp, pw)
    h2, hs2 = _tc_dense2(agg1, hs1, x, dv, gcn_W1, gcn_b1.reshape(1, -1),
                         ln_g1.reshape(1, -1), ln_b1.reshape(1, -1), blk)

    agg2 = _sc_edge(hs2, comb, ewp, pw)
    return _tc_dense3(agg2, hs2, h2, dv, gcn_W2, gcn_b2.reshape(1, -1),
                      ln_g2.reshape(1, -1), ln_b2.reshape(1, -1),
                      z_enc, aggr_W[:128], aggr_W[128:],
                      aggr_b.reshape(1, -1), blk)


# speedup vs baseline: 1.0279x; 1.0279x over previous
import functools

import jax
import jax.numpy as jnp
from jax import lax
from jax.experimental import pallas as pl
from jax.experimental.pallas import tpu as pltpu
from jax.experimental.pallas import tpu_sc as plsc


NC = 2
NS = 16
NW = NC * NS
CH = 64


def _sc_deg_body(n_out, nchd, dstp, ewp, out, idx_d, ew_v, zbuf, deg_sh):
    c = lax.axis_index("c")
    s = lax.axis_index("s")
    wid = s * NC + c

    pltpu.sync_copy(dstp.at[wid], idx_d)
    pltpu.sync_copy(ewp.at[wid], ew_v)

    @pl.when(s == 0)
    def _():
        z16 = jnp.zeros((16,), jnp.float32)

        @pl.loop(0, 128)
        def _z(i):
            zbuf[pl.ds(i * 16, 16)] = z16

        @pl.loop(0, n_out // 2048)
        def _zz(i):
            pltpu.sync_copy(zbuf, deg_sh.at[pl.ds(i * 2048, 2048)])

    plsc.subcore_barrier()

    @pl.loop(0, nchd)
    def _edges(ch):
        pltpu.sync_copy(ew_v.at[ch], deg_sh.at[idx_d.at[ch]], add=True)

    plsc.subcore_barrier()

    @pl.when(s == 0)
    def _():
        pltpu.sync_copy(deg_sh, out.at[c])


def _sc_deg(dstp, ewp, n_nodes):
    n_out = -(-n_nodes // 2048) * 2048
    nchd = dstp.shape[1]
    body = functools.partial(_sc_deg_body, n_out, nchd)
    return pl.kernel(
        body,
        out_type=jax.ShapeDtypeStruct((NC, n_out), jnp.float32),
        mesh=plsc.VectorSubcoreMesh(core_axis_name="c", subcore_axis_name="s",
                                    num_cores=NC, num_subcores=NS),
        scratch_types=[
            pltpu.VMEM((nchd, 128), jnp.int32),
            pltpu.VMEM((nchd, 128), jnp.float32),
            pltpu.VMEM((2048,), jnp.float32),
            pltpu.VMEM_SHARED((n_out,), jnp.float32),
        ],
    )(dstp, ewp)


def _sc_edge_body(n_pad, pw, hs, srcp, dstp, ewp, out, *refs):
    nch = pw // CH
    c = lax.axis_index("c")
    s = lax.axis_index("s")
    wid = s * NC + c
    base = wid * pw
    iss = refs[0:8]
    ids = refs[8:16]
    ews = refs[16:24]
    rows = refs[24:28]
    sis = refs[28:36]
    sgs = refs[36:40]
    sss = refs[40:44]
    agg_sh = refs[44]
    r0 = rows[0]

    rpt = n_pad // NS
    zeros16 = jnp.zeros((16,), jnp.float32)
    for e in range(8):
        for k in range(8):
            r0[e, pl.ds(k * 16, 16)] = zeros16

    @pl.loop(0, rpt // 8)
    def _zero(i):
        pltpu.sync_copy(r0.at[pl.ds(0, 8)],
                        agg_sh.at[pl.ds(s * rpt + i * 8, 8)])

    plsc.subcore_barrier()

    def stage_descs(ch, m):
        off = base + ch * CH
        return [
            pltpu.make_async_copy(srcp.at[pl.ds(off, CH)], iss[m], sis[m]),
            pltpu.make_async_copy(dstp.at[pl.ds(off, CH)], ids[m], sis[m]),
            pltpu.make_async_copy(ewp.at[pl.ds(off, CH)], ews[m], sis[m]),
        ]

    def stage(ch, m):
        for d in stage_descs(ch, m):
            d.start()

    def wait_stage(ch, m):
        for d in stage_descs(ch, m):
            d.wait()

    def gather(m, b):
        pltpu.make_async_copy(hs.at[iss[m]], rows[b], sgs[b]).start()

    def wait_gather(m, b):
        pltpu.make_async_copy(hs.at[iss[m]], rows[b], sgs[b]).wait()

    def scale(m, b):
        buf = rows[b]
        ewb = ews[m]

        @pl.loop(0, CH // 16)
        def _h(h):
            wv = ewb[pl.ds(h * 16, 16)]
            for i in range(16):
                w = wv[i]
                for k in range(8):
                    sl = buf[h * 16 + i, pl.ds(k * 16, 16)]
                    buf[h * 16 + i, pl.ds(k * 16, 16)] = sl * w

    def scatter(m, b):
        pltpu.make_async_copy(rows[b], agg_sh.at[ids[m]],
                              sss[b]).start(add=True)

    def wait_scatter(m, b):
        pltpu.make_async_copy(rows[b], agg_sh.at[ids[m]], sss[b]).wait()

    def body(ch, m):
        b = m % 4
        b2 = (m + 2) % 4
        m2 = (m + 2) % 8
        m4 = (m + 4) % 8
        m6 = (m + 6) % 8

        @pl.when(jnp.logical_and(ch >= 2, ch + 2 < nch))
        def _():
            wait_scatter(m6, b2)

        @pl.when(ch + 2 < nch)
        def _():
            wait_stage(ch + 2, m2)
            gather(m2, b2)

        @pl.when(ch + 4 < nch)
        def _():
            stage(ch + 4, m4)

        wait_gather(m, b)
        scale(m, b)
        scatter(m, b)

    for ch0 in range(4):
        stage(ch0, ch0)
    wait_stage(0, 0)
    gather(0, 0)
    wait_stage(1, 1)
    gather(1, 1)

    @pl.loop(0, nch, step=8)
    def _main(j):
        for m in range(8):
            body(j + m, m)

    for t in range(4, 0, -1):
        wait_scatter((nch - t) % 8, (nch - t) % 4)

    plsc.subcore_barrier()

    pltpu.sync_copy(agg_sh.at[pl.ds(s * rpt, rpt)],
                    out.at[c].at[pl.ds(s * rpt, rpt)])


def _sc_edge(hs, srcp, dstp, ewp, pw):
    n_pad = -(-hs.shape[0] // (NS * 8)) * (NS * 8)
    body = functools.partial(_sc_edge_body, n_pad, pw)
    idx_t = [pltpu.VMEM((CH,), jnp.int32)] * 16
    ew_t = [pltpu.VMEM((CH,), jnp.float32)] * 8
    row_t = [pltpu.VMEM((CH, 128), jnp.float32)] * 4
    sem_t = [pltpu.SemaphoreType.DMA] * 16
    return pl.kernel(
        body,
        out_type=jax.ShapeDtypeStruct((NC, n_pad, 128), jnp.float32),
        mesh=plsc.VectorSubcoreMesh(core_axis_name="c", subcore_axis_name="s",
                                    num_cores=NC, num_subcores=NS),
        scratch_types=idx_t + ew_t + row_t + sem_t + [
            pltpu.VMEM_SHARED((n_pad, 128), jnp.float32),
        ],
    )(hs, srcp, dstp, ewp)


def _enc_body(z_ref, w1_ref, b1_ref, w2_ref, b2_ref, out_ref):
    h = jnp.dot(z_ref[...], w1_ref[...], preferred_element_type=jnp.float32)
    h = jnp.maximum(h + b1_ref[...], 0.0)
    out_ref[...] = (
        jnp.dot(h, w2_ref[...], preferred_element_type=jnp.float32)
        + b2_ref[...]
    )


def _tc_enc(z, w1, b1, w2, b2):
    return pl.pallas_call(
        _enc_body,
        out_shape=jax.ShapeDtypeStruct((z.shape[0], w2.shape[1]), jnp.float32),
    )(z, w1, b1, w2, b2)


def _dense1_body(blk, x_ref, dp_ref, hs_ref, dv_ref):
    i = pl.program_id(0)
    dp = dp_ref[:, pl.ds(i * blk, blk)]
    d = 1.0 + jnp.sum(dp, axis=0)
    dinv = lax.rsqrt(d)[:, None]
    rep = jnp.broadcast_to(dinv, x_ref.shape)
    dv_ref[...] = rep
    hs_ref[...] = x_ref[...] * rep


def _tc_dense1(x, deg_part, blk):
    n = x.shape[0]
    grid = -(-n // blk)
    return pl.pallas_call(
        functools.partial(_dense1_body, blk),
        grid=(grid,),
        in_specs=[
            pl.BlockSpec((blk, 128), lambda i: (i, 0)),
            pl.BlockSpec(deg_part.shape, lambda i: (0, 0)),
        ],
        out_specs=[
            pl.BlockSpec((blk, 128), lambda i: (i, 0)),
            pl.BlockSpec((blk, 128), lambda i: (i, 0)),
        ],
        out_shape=[
            jax.ShapeDtypeStruct((n, 128), jnp.float32),
            jax.ShapeDtypeStruct((n, 128), jnp.float32),
        ],
    )(x, deg_part)


def _ln(y, g_ref, b_ref):
    m = jnp.mean(y, axis=-1, keepdims=True)
    cdev = y - m
    v = jnp.mean(cdev * cdev, axis=-1, keepdims=True)
    return cdev * lax.rsqrt(v + 1e-5) * g_ref[...] + b_ref[...]


def _dense2_body(agg_ref, hs_ref, x_ref, dv_ref, w_ref, b_ref, g_ref,
                 bl_ref, h2_ref, hs2_ref):
    a = (agg_ref[0] + agg_ref[1] + hs_ref[...]) * dv_ref[...]
    y = jnp.dot(a, w_ref[...], preferred_element_type=jnp.float32) + b_ref[...]
    y = jnp.maximum(_ln(y, g_ref, bl_ref), 0.0)
    h2 = y + x_ref[...]
    h2_ref[...] = h2
    hs2_ref[...] = h2 * dv_ref[...]


def _tc_dense2(agg, hs, x, dv, w, b, g, bl, blk):
    n = x.shape[0]
    grid = -(-n // blk)
    return pl.pallas_call(
        _dense2_body,
        grid=(grid,),
        in_specs=[
            pl.BlockSpec((NC, blk, 128), lambda i: (0, i, 0)),
            pl.BlockSpec((blk, 128), lambda i: (i, 0)),
            pl.BlockSpec((blk, 128), lambda i: (i, 0)),
            pl.BlockSpec((blk, 128), lambda i: (i, 0)),
            pl.BlockSpec((128, 128), lambda i: (0, 0)),
            pl.BlockSpec((1, 128), lambda i: (0, 0)),
            pl.BlockSpec((1, 128), lambda i: (0, 0)),
            pl.BlockSpec((1, 128), lambda i: (0, 0)),
        ],
        out_specs=[
            pl.BlockSpec((blk, 128), lambda i: (i, 0)),
            pl.BlockSpec((blk, 128), lambda i: (i, 0)),
        ],
        out_shape=[
            jax.ShapeDtypeStruct((n, 128), jnp.float32),
            jax.ShapeDtypeStruct((n, 128), jnp.float32),
        ],
    )(agg, hs, x, dv, w, b, g, bl)


def _dense3_body(n, blk, grid, agg_ref, hs_ref, h_ref, dv_ref, w_ref, b_ref,
                 g_ref, bl_ref, z_ref, wt_ref, wb_ref, ab_ref, out_ref,
                 gsum_ref):
    i = pl.program_id(0)
    a = (agg_ref[0] + agg_ref[1] + hs_ref[...]) * dv_ref[...]
    y = jnp.dot(a, w_ref[...], preferred_element_type=jnp.float32) + b_ref[...]
    gblk = _ln(y, g_ref, bl_ref) + h_ref[...]
    rid = lax.broadcasted_iota(jnp.int32, gblk.shape, 0)
    gblk = jnp.where(rid < n - i * blk, gblk, 0.0)

    @pl.when(i == 0)
    def _():
        gsum_ref[...] = jnp.zeros_like(gsum_ref)

    gsum_ref[...] += jnp.sum(gblk, axis=0, keepdims=True)

    @pl.when(i == grid - 1)
    def _():
        gm = gsum_ref[...] * (1.0 / n)
        gterm = jnp.dot(gm, wb_ref[...], preferred_element_type=jnp.float32)
        out_ref[...] = (
            jnp.dot(z_ref[...], wt_ref[...],
                    preferred_element_type=jnp.float32)
            + gterm + ab_ref[...]
        )


def _tc_dense3(agg, hs, h, dv, w, b, g, bl, z_enc, wt, wb, ab, blk):
    n = h.shape[0]
    grid = -(-n // blk)
    nb = z_enc.shape[0]
    return pl.pallas_call(
        functools.partial(_dense3_body, n, blk, grid),
        grid=(grid,),
        in_specs=[
            pl.BlockSpec((NC, blk, 128), lambda i: (0, i, 0)),
            pl.BlockSpec((blk, 128), lambda i: (i, 0)),
            pl.BlockSpec((blk, 128), lambda i: (i, 0)),
            pl.BlockSpec((blk, 128), lambda i: (i, 0)),
            pl.BlockSpec((128, 128), lambda i: (0, 0)),
            pl.BlockSpec((1, 128), lambda i: (0, 0)),
            pl.BlockSpec((1, 128), lambda i: (0, 0)),
            pl.BlockSpec((1, 128), lambda i: (0, 0)),
            pl.BlockSpec((nb, 128), lambda i: (0, 0)),
            pl.BlockSpec((128, 128), lambda i: (0, 0)),
            pl.BlockSpec((128, 128), lambda i: (0, 0)),
            pl.BlockSpec((1, 128), lambda i: (0, 0)),
        ],
        out_specs=pl.BlockSpec((nb, 128), lambda i: (0, 0)),
        out_shape=jax.ShapeDtypeStruct((nb, 128), jnp.float32),
        scratch_shapes=[pltpu.VMEM((1, 128), jnp.float32)],
    )(agg, hs, h, dv, w, b, g, bl, z_enc, wt, wb, ab)


def kernel(z, x, edge_index, edge_attr, enc_W1, enc_b1, enc_W2, enc_b2,
           gcn_W1, gcn_b1, ln_g1, ln_b1, gcn_W2, gcn_b2, ln_g2, ln_b2,
           aggr_W, aggr_b):
    n = x.shape[0]
    e = edge_index.shape[1]

    pw = -(-e // NW)
    pw = -(-pw // (8 * CH)) * (8 * CH)
    ep = pw * NW
    nch = pw // CH
    pad = ep - e

    src = edge_index[0].astype(jnp.int32)
    dst = edge_index[1].astype(jnp.int32)
    pidx = (jnp.arange(pad, dtype=jnp.int32) * 97) % n
    srcp = jnp.concatenate([src, pidx])
    dstp = jnp.concatenate([dst, pidx])
    ewp = jnp.concatenate(
        [edge_attr.astype(jnp.float32), jnp.zeros((pad,), jnp.float32)]
    )
    dstpd = dstp.reshape(NW, pw // 128, 128)
    ewpd = ewp.reshape(NW, pw // 128, 128)

    blk = 2048

    z_enc = _tc_enc(z, enc_W1, enc_b1.reshape(1, -1), enc_W2,
                    enc_b2.reshape(1, -1))

    deg_part = _sc_deg(dstpd, ewpd, n)
    hs1, dv = _tc_dense1(x, deg_part, blk)

    agg1 = _sc_edge(hs1, srcp, dstp, ewp, pw)
    h2, hs2 = _tc_dense2(agg1, hs1, x, dv, gcn_W1, gcn_b1.reshape(1, -1),
                         ln_g1.reshape(1, -1), ln_b1.reshape(1, -1), blk)

    agg2 = _sc_edge(hs2, srcp, dstp, ewp, pw)
    return _tc_dense3(agg2, hs2, h2, dv, gcn_W2, gcn_b2.reshape(1, -1),
                      ln_g2.reshape(1, -1), ln_b2.reshape(1, -1),
                      z_enc, aggr_W[:128], aggr_W[128:],
                      aggr_b.reshape(1, -1), blk)


# speedup vs baseline: 1.0588x; 1.0301x over previous
import functools

import jax
import jax.numpy as jnp
from jax import lax
from jax.experimental import pallas as pl
from jax.experimental.pallas import tpu as pltpu
from jax.experimental.pallas import tpu_sc as plsc


NC = 2
NS = 16
NW = NC * NS
CH = 80


def _sc_deg_body(n_out, nchd, dstp, ewp, out, idx_d, ew_v, zbuf, deg_sh):
    c = lax.axis_index("c")
    s = lax.axis_index("s")
    wid = s * NC + c

    pltpu.sync_copy(dstp.at[wid], idx_d)
    pltpu.sync_copy(ewp.at[wid], ew_v)

    @pl.when(s == 0)
    def _():
        z16 = jnp.zeros((16,), jnp.float32)

        @pl.loop(0, 128)
        def _z(i):
            zbuf[pl.ds(i * 16, 16)] = z16

        @pl.loop(0, n_out // 2048)
        def _zz(i):
            pltpu.sync_copy(zbuf, deg_sh.at[pl.ds(i * 2048, 2048)])

    plsc.subcore_barrier()

    @pl.loop(0, nchd)
    def _edges(ch):
        pltpu.sync_copy(ew_v.at[ch], deg_sh.at[idx_d.at[ch]], add=True)

    plsc.subcore_barrier()

    @pl.when(s == 0)
    def _():
        pltpu.sync_copy(deg_sh, out.at[c])


def _sc_deg(dstp, ewp, n_nodes):
    n_out = -(-n_nodes // 2048) * 2048
    nchd = dstp.shape[1]
    body = functools.partial(_sc_deg_body, n_out, nchd)
    return pl.kernel(
        body,
        out_type=jax.ShapeDtypeStruct((NC, n_out), jnp.float32),
        mesh=plsc.VectorSubcoreMesh(core_axis_name="c", subcore_axis_name="s",
                                    num_cores=NC, num_subcores=NS),
        scratch_types=[
            pltpu.VMEM((nchd, 128), jnp.int32),
            pltpu.VMEM((nchd, 128), jnp.float32),
            pltpu.VMEM((2048,), jnp.float32),
            pltpu.VMEM_SHARED((n_out,), jnp.float32),
        ],
    )(dstp, ewp)


def _sc_edge_body(n_pad, pw, hs, srcp, dstp, ewp, out, *refs):
    nch = pw // CH
    c = lax.axis_index("c")
    s = lax.axis_index("s")
    wid = s * NC + c
    base = wid * pw
    iss = refs[0:8]
    ids = refs[8:16]
    ews = refs[16:24]
    rows = refs[24:28]
    sis = refs[28:36]
    sgs = refs[36:40]
    sss = refs[40:44]
    agg_sh = refs[44]
    r0 = rows[0]

    rpt = n_pad // NS
    zeros16 = jnp.zeros((16,), jnp.float32)
    for e in range(8):
        for k in range(8):
            r0[e, pl.ds(k * 16, 16)] = zeros16

    @pl.loop(0, rpt // 8)
    def _zero(i):
        pltpu.sync_copy(r0.at[pl.ds(0, 8)],
                        agg_sh.at[pl.ds(s * rpt + i * 8, 8)])

    plsc.subcore_barrier()

    def stage_descs(ch, m):
        off = base + ch * CH
        return [
            pltpu.make_async_copy(srcp.at[pl.ds(off, CH)], iss[m], sis[m]),
            pltpu.make_async_copy(dstp.at[pl.ds(off, CH)], ids[m], sis[m]),
            pltpu.make_async_copy(ewp.at[pl.ds(off, CH)], ews[m], sis[m]),
        ]

    def stage(ch, m):
        for d in stage_descs(ch, m):
            d.start()

    def wait_stage(ch, m):
        for d in stage_descs(ch, m):
            d.wait()

    def gather(m, b):
        pltpu.make_async_copy(hs.at[iss[m]], rows[b], sgs[b]).start()

    def wait_gather(m, b):
        pltpu.make_async_copy(hs.at[iss[m]], rows[b], sgs[b]).wait()

    def scale(m, b):
        buf = rows[b]
        ewb = ews[m]

        @pl.loop(0, CH // 16)
        def _h(h):
            wv = ewb[pl.ds(h * 16, 16)]
            for i in range(16):
                w = wv[i]
                for k in range(8):
                    sl = buf[h * 16 + i, pl.ds(k * 16, 16)]
                    buf[h * 16 + i, pl.ds(k * 16, 16)] = sl * w

    def scatter(m, b):
        pltpu.make_async_copy(rows[b], agg_sh.at[ids[m]],
                              sss[b]).start(add=True)

    def wait_scatter(m, b):
        pltpu.make_async_copy(rows[b], agg_sh.at[ids[m]], sss[b]).wait()

    def body(ch, m):
        b = m % 4
        b2 = (m + 2) % 4
        m2 = (m + 2) % 8
        m4 = (m + 4) % 8
        m6 = (m + 6) % 8

        @pl.when(jnp.logical_and(ch >= 2, ch + 2 < nch))
        def _():
            wait_scatter(m6, b2)

        @pl.when(ch + 2 < nch)
        def _():
            wait_stage(ch + 2, m2)
            gather(m2, b2)

        @pl.when(ch + 4 < nch)
        def _():
            stage(ch + 4, m4)

        wait_gather(m, b)
        scale(m, b)
        scatter(m, b)

    for ch0 in range(4):
        stage(ch0, ch0)
    wait_stage(0, 0)
    gather(0, 0)
    wait_stage(1, 1)
    gather(1, 1)

    @pl.loop(0, nch, step=8)
    def _main(j):
        for m in range(8):
            body(j + m, m)

    for t in range(4, 0, -1):
        wait_scatter((nch - t) % 8, (nch - t) % 4)

    plsc.subcore_barrier()

    pltpu.sync_copy(agg_sh.at[pl.ds(s * rpt, rpt)],
                    out.at[c].at[pl.ds(s * rpt, rpt)])


def _sc_edge(hs, srcp, dstp, ewp, pw):
    n_pad = -(-hs.shape[0] // (NS * 8)) * (NS * 8)
    body = functools.partial(_sc_edge_body, n_pad, pw)
    idx_t = [pltpu.VMEM((CH,), jnp.int32)] * 16
    ew_t = [pltpu.VMEM((CH,), jnp.float32)] * 8
    row_t = [pltpu.VMEM((CH, 128), jnp.float32)] * 4
    sem_t = [pltpu.SemaphoreType.DMA] * 16
    return pl.kernel(
        body,
        out_type=jax.ShapeDtypeStruct((NC, n_pad, 128), jnp.float32),
        mesh=plsc.VectorSubcoreMesh(core_axis_name="c", subcore_axis_name="s",
                                    num_cores=NC, num_subcores=NS),
        scratch_types=idx_t + ew_t + row_t + sem_t + [
            pltpu.VMEM_SHARED((n_pad, 128), jnp.float32),
        ],
    )(hs, srcp, dstp, ewp)


def _enc_body(z_ref, w1_ref, b1_ref, w2_ref, b2_ref, out_ref):
    h = jnp.dot(z_ref[...], w1_ref[...], preferred_element_type=jnp.float32)
    h = jnp.maximum(h + b1_ref[...], 0.0)
    out_ref[...] = (
        jnp.dot(h, w2_ref[...], preferred_element_type=jnp.float32)
        + b2_ref[...]
    )


def _tc_enc(z, w1, b1, w2, b2):
    return pl.pallas_call(
        _enc_body,
        out_shape=jax.ShapeDtypeStruct((z.shape[0], w2.shape[1]), jnp.float32),
    )(z, w1, b1, w2, b2)


def _dense1_body(blk, x_ref, dp_ref, hs_ref, dv_ref):
    i = pl.program_id(0)
    dp = dp_ref[:, pl.ds(i * blk, blk)]
    d = 1.0 + jnp.sum(dp, axis=0)
    dinv = lax.rsqrt(d)[:, None]
    rep = jnp.broadcast_to(dinv, x_ref.shape)
    dv_ref[...] = rep
    hs_ref[...] = x_ref[...] * rep


def _tc_dense1(x, deg_part, blk):
    n = x.shape[0]
    grid = -(-n // blk)
    return pl.pallas_call(
        functools.partial(_dense1_body, blk),
        grid=(grid,),
        in_specs=[
            pl.BlockSpec((blk, 128), lambda i: (i, 0)),
            pl.BlockSpec(deg_part.shape, lambda i: (0, 0)),
        ],
        out_specs=[
            pl.BlockSpec((blk, 128), lambda i: (i, 0)),
            pl.BlockSpec((blk, 128), lambda i: (i, 0)),
        ],
        out_shape=[
            jax.ShapeDtypeStruct((n, 128), jnp.float32),
            jax.ShapeDtypeStruct((n, 128), jnp.float32),
        ],
    )(x, deg_part)


def _ln(y, g_ref, b_ref):
    m = jnp.mean(y, axis=-1, keepdims=True)
    cdev = y - m
    v = jnp.mean(cdev * cdev, axis=-1, keepdims=True)
    return cdev * lax.rsqrt(v + 1e-5) * g_ref[...] + b_ref[...]


def _dense2_body(agg_ref, hs_ref, x_ref, dv_ref, w_ref, b_ref, g_ref,
                 bl_ref, h2_ref, hs2_ref):
    a = (agg_ref[0] + agg_ref[1] + hs_ref[...]) * dv_ref[...]
    y = jnp.dot(a, w_ref[...], preferred_element_type=jnp.float32) + b_ref[...]
    y = jnp.maximum(_ln(y, g_ref, bl_ref), 0.0)
    h2 = y + x_ref[...]
    h2_ref[...] = h2
    hs2_ref[...] = h2 * dv_ref[...]


def _tc_dense2(agg, hs, x, dv, w, b, g, bl, blk):
    n = x.shape[0]
    grid = -(-n // blk)
    return pl.pallas_call(
        _dense2_body,
        grid=(grid,),
        in_specs=[
            pl.BlockSpec((NC, blk, 128), lambda i: (0, i, 0)),
            pl.BlockSpec((blk, 128), lambda i: (i, 0)),
            pl.BlockSpec((blk, 128), lambda i: (i, 0)),
            pl.BlockSpec((blk, 128), lambda i: (i, 0)),
            pl.BlockSpec((128, 128), lambda i: (0, 0)),
            pl.BlockSpec((1, 128), lambda i: (0, 0)),
            pl.BlockSpec((1, 128), lambda i: (0, 0)),
            pl.BlockSpec((1, 128), lambda i: (0, 0)),
        ],
        out_specs=[
            pl.BlockSpec((blk, 128), lambda i: (i, 0)),
            pl.BlockSpec((blk, 128), lambda i: (i, 0)),
        ],
        out_shape=[
            jax.ShapeDtypeStruct((n, 128), jnp.float32),
            jax.ShapeDtypeStruct((n, 128), jnp.float32),
        ],
    )(agg, hs, x, dv, w, b, g, bl)


def _dense3_body(n, blk, grid, agg_ref, hs_ref, h_ref, dv_ref, w_ref, b_ref,
                 g_ref, bl_ref, z_ref, wt_ref, wb_ref, ab_ref, out_ref,
                 gsum_ref):
    i = pl.program_id(0)
    a = (agg_ref[0] + agg_ref[1] + hs_ref[...]) * dv_ref[...]
    y = jnp.dot(a, w_ref[...], preferred_element_type=jnp.float32) + b_ref[...]
    gblk = _ln(y, g_ref, bl_ref) + h_ref[...]
    rid = lax.broadcasted_iota(jnp.int32, gblk.shape, 0)
    gblk = jnp.where(rid < n - i * blk, gblk, 0.0)

    @pl.when(i == 0)
    def _():
        gsum_ref[...] = jnp.zeros_like(gsum_ref)

    gsum_ref[...] += jnp.sum(gblk, axis=0, keepdims=True)

    @pl.when(i == grid - 1)
    def _():
        gm = gsum_ref[...] * (1.0 / n)
        gterm = jnp.dot(gm, wb_ref[...], preferred_element_type=jnp.float32)
        out_ref[...] = (
            jnp.dot(z_ref[...], wt_ref[...],
                    preferred_element_type=jnp.float32)
            + gterm + ab_ref[...]
        )


def _tc_dense3(agg, hs, h, dv, w, b, g, bl, z_enc, wt, wb, ab, blk):
    n = h.shape[0]
    grid = -(-n // blk)
    nb = z_enc.shape[0]
    return pl.pallas_call(
        functools.partial(_dense3_body, n, blk, grid),
        grid=(grid,),
        in_specs=[
            pl.BlockSpec((NC, blk, 128), lambda i: (0, i, 0)),
            pl.BlockSpec((blk, 128), lambda i: (i, 0)),
            pl.BlockSpec((blk, 128), lambda i: (i, 0)),
            pl.BlockSpec((blk, 128), lambda i: (i, 0)),
            pl.BlockSpec((128, 128), lambda i: (0, 0)),
            pl.BlockSpec((1, 128), lambda i: (0, 0)),
            pl.BlockSpec((1, 128), lambda i: (0, 0)),
            pl.BlockSpec((1, 128), lambda i: (0, 0)),
            pl.BlockSpec((nb, 128), lambda i: (0, 0)),
            pl.BlockSpec((128, 128), lambda i: (0, 0)),
            pl.BlockSpec((128, 128), lambda i: (0, 0)),
            pl.BlockSpec((1, 128), lambda i: (0, 0)),
        ],
        out_specs=pl.BlockSpec((nb, 128), lambda i: (0, 0)),
        out_shape=jax.ShapeDtypeStruct((nb, 128), jnp.float32),
        scratch_shapes=[pltpu.VMEM((1, 128), jnp.float32)],
    )(agg, hs, h, dv, w, b, g, bl, z_enc, wt, wb, ab)


def kernel(z, x, edge_index, edge_attr, enc_W1, enc_b1, enc_W2, enc_b2,
           gcn_W1, gcn_b1, ln_g1, ln_b1, gcn_W2, gcn_b2, ln_g2, ln_b2,
           aggr_W, aggr_b):
    n = x.shape[0]
    e = edge_index.shape[1]

    pw = -(-e // NW)
    pw = -(-pw // (8 * CH)) * (8 * CH)
    ep = pw * NW
    nch = pw // CH
    pad = ep - e

    src = edge_index[0].astype(jnp.int32)
    dst = edge_index[1].astype(jnp.int32)
    pidx = (jnp.arange(pad, dtype=jnp.int32) * 97) % n
    srcp = jnp.concatenate([src, pidx])
    dstp = jnp.concatenate([dst, pidx])
    ewp = jnp.concatenate(
        [edge_attr.astype(jnp.float32), jnp.zeros((pad,), jnp.float32)]
    )
    dstpd = dstp.reshape(NW, pw // 128, 128)
    ewpd = ewp.reshape(NW, pw // 128, 128)

    blk = 2048

    z_enc = _tc_enc(z, enc_W1, enc_b1.reshape(1, -1), enc_W2,
                    enc_b2.reshape(1, -1))

    deg_part = _sc_deg(dstpd, ewpd, n)
    hs1, dv = _tc_dense1(x, deg_part, blk)

    agg1 = _sc_edge(hs1, srcp, dstp, ewp, pw)
    h2, hs2 = _tc_dense2(agg1, hs1, x, dv, gcn_W1, gcn_b1.reshape(1, -1),
                         ln_g1.reshape(1, -1), ln_b1.reshape(1, -1), blk)

    agg2 = _sc_edge(hs2, srcp, dstp, ewp, pw)
    return _tc_dense3(agg2, hs2, h2, dv, gcn_W2, gcn_b2.reshape(1, -1),
                      ln_g2.reshape(1, -1), ln_b2.reshape(1, -1),
                      z_enc, aggr_W[:128], aggr_W[128:],
                      aggr_b.reshape(1, -1), blk)
